# baseline (speedup 1.0000x reference)
import jax
import jax.numpy as jnp
from jax import lax
from jax.experimental import pallas as pl
from jax.experimental.pallas import tpu as pltpu
from jax.experimental.pallas import tpu_sc as plsc

H = 2048
W = 2048
C = 3
N = 1048576

NUM_WORKERS = 32
PTS_PER_TILE = N // NUM_WORKERS
CHUNK = 128
G = PTS_PER_TILE // CHUNK
L = 16
D = 8
NTEX = H * W


def _body(xs_hbm, table_hbm, out_hbm,
          xs_all,
          it0, it10, ib0, ib10, wx0, wy0, gt0, gt10, gb0, gb10,
          it1, it11, ib1, ib11, wx1, wy1, gt1, gt11, gb1, gb11,
          obuf, sem0, sem1):
  wid = lax.axis_index("s") * 2 + lax.axis_index("c")
  lane = lax.broadcasted_iota(jnp.int32, (L,), 0)
  slots = (
      (it0, it10, ib0, ib10, wx0, wy0, gt0, gt10, gb0, gb10, sem0),
      (it1, it11, ib1, ib11, wx1, wy1, gt1, gt11, gb1, gb11, sem1),
  )

  pltpu.sync_copy(xs_hbm.at[pl.ds(2 * wid * PTS_PER_TILE, 2 * PTS_PER_TILE)],
                  xs_all)

  def pass1(g, slot):
    idxt, idxt1, idxb, idxb1, wx_ref, wy_ref = slot[:6]
    for q in range(CHUNK // L):
      pbase = q * L
      ex = 2 * (g * CHUNK + pbase + lane)
      px = plsc.load_gather(xs_all, [ex])
      py = plsc.load_gather(xs_all, [ex + 1])
      sx = px * jnp.float32(W)
      sy = py * jnp.float32(H)
      ix = sx.astype(jnp.int32)
      iy = sy.astype(jnp.int32)
      wx = sx - ix.astype(jnp.float32)
      wy = sy - iy.astype(jnp.float32)
      x0 = jnp.minimum(jnp.maximum(ix, 0), W - 1)
      y0 = jnp.minimum(jnp.maximum(iy, 0), H - 1)
      y1 = jnp.minimum(y0 + 1, H - 1)
      wx = jnp.where(x0 >= W - 1, jnp.float32(0.0), wx)
      kt = y0 * W + x0
      kb = y1 * W + x0
      sl = pl.ds(pbase, L)
      idxt[sl] = kt
      idxt1[sl] = kt + 1
      idxb[sl] = kb
      idxb1[sl] = kb + 1
      wx_ref[sl] = wx
      wy_ref[sl] = wy

  def fire(slot):
    idxt, idxt1, idxb, idxb1 = slot[:4]
    gt, gt1, gb, gb1, sem = slot[6:11]
    pltpu.async_copy(table_hbm.at[idxt], gt, sem)
    pltpu.async_copy(table_hbm.at[idxt1], gt1, sem)
    pltpu.async_copy(table_hbm.at[idxb], gb, sem)
    pltpu.async_copy(table_hbm.at[idxb1], gb1, sem)

  def drain(slot):
    idxt, idxt1, idxb, idxb1 = slot[:4]
    gt, gt1, gb, gb1, sem = slot[6:11]
    pltpu.make_async_copy(table_hbm.at[idxt], gt, sem).wait()
    pltpu.make_async_copy(table_hbm.at[idxt1], gt1, sem).wait()
    pltpu.make_async_copy(table_hbm.at[idxb], gb, sem).wait()
    pltpu.make_async_copy(table_hbm.at[idxb1], gb1, sem).wait()

  def combine(slot, b):
    wx_ref, wy_ref, gt, gt1, gb, gb1 = slot[4:10]
    for q in range(CHUNK // L):
      pbase = q * L
      sl = pl.ds(pbase, L)
      wx = wx_ref[sl]
      wy = wy_ref[sl]
      prow = pbase + lane
      obase = b * (C * CHUNK)
      for c in range(C):
        ccol = jnp.full((L,), c, jnp.int32)
        t0 = plsc.load_gather(gt, [prow, ccol])
        t1 = plsc.load_gather(gt1, [prow, ccol])
        b0 = plsc.load_gather(gb, [prow, ccol])
        b1 = plsc.load_gather(gb1, [prow, ccol])
        top = t0 + wx * (t1 - t0)
        bot = b0 + wx * (b1 - b0)
        o = top + wy * (bot - top)
        plsc.store_scatter(obuf, [obase + 3 * prow + c], o)

  for b in (0, 1):
    pass1(jnp.int32(b), slots[b])
    fire(slots[b])

  def body(i, carry):
    for b in (0, 1):
      g = 2 * i + b
      drain(slots[b])
      combine(slots[b], b)
      gn = g + 2
      gn = jnp.where(gn >= G, gn - G, gn)
      pass1(gn, slots[b])
      fire(slots[b])
    pltpu.sync_copy(
        obuf, out_hbm.at[pl.ds(3 * (wid * PTS_PER_TILE + 2 * i * CHUNK),
                               2 * C * CHUNK)])
    return carry

  lax.fori_loop(0, G // 2, body, 0)
  drain(slots[0])
  drain(slots[1])


@jax.jit
def _run(xs_flat, table):
  mesh = plsc.VectorSubcoreMesh(core_axis_name="c", subcore_axis_name="s")
  slot_types = [
      pltpu.VMEM((CHUNK,), jnp.int32),
      pltpu.VMEM((CHUNK,), jnp.int32),
      pltpu.VMEM((CHUNK,), jnp.int32),
      pltpu.VMEM((CHUNK,), jnp.int32),
      pltpu.VMEM((CHUNK,), jnp.float32),
      pltpu.VMEM((CHUNK,), jnp.float32),
      pltpu.VMEM((CHUNK, D), jnp.float32),
      pltpu.VMEM((CHUNK, D), jnp.float32),
      pltpu.VMEM((CHUNK, D), jnp.float32),
      pltpu.VMEM((CHUNK, D), jnp.float32),
  ]
  kern = pl.kernel(
      _body,
      out_type=jax.ShapeDtypeStruct((N * C,), jnp.float32),
      mesh=mesh,
      compiler_params=pltpu.CompilerParams(
          needs_layout_passes=False, use_tc_tiling_on_sc=False),
      scratch_types=(
          [pltpu.VMEM((2 * PTS_PER_TILE,), jnp.float32)]
          + slot_types + slot_types
          + [pltpu.VMEM((2 * C * CHUNK,), jnp.float32),
             pltpu.SemaphoreType.DMA,
             pltpu.SemaphoreType.DMA]
      ),
  )
  return kern(xs_flat, table)


def kernel(xs, data):
  rows = data.reshape(NTEX, C)
  table = jnp.pad(rows, ((0, 1), (0, D - C)))
  out_flat = _run(xs.reshape(-1), table)
  return out_flat.reshape(N, C)
